# baseline (speedup 1.0000x reference)
import functools

import jax
import jax.numpy as jnp
from jax import lax
from jax.experimental import pallas as pl
from jax.experimental.pallas import tpu as pltpu
from jax.experimental.pallas import tpu_sc as plsc

N_NODES = 10000
D_FEAT = 128
N_EDGES = 320000

NC = 2
NS = 16
NW = NC * NS
EPW = N_EDGES // NW
B_EDGE = 80
NB = EPW // B_EDGE
ROWS_PER_TILE = 624
ROWS_TAIL = N_NODES - NS * ROWS_PER_TILE


def _segment_sum_sc(x, src3, dst3, zeros):
    mesh = plsc.VectorSubcoreMesh(core_axis_name="c", subcore_axis_name="s")

    @functools.partial(
        pl.kernel,
        mesh=mesh,
        out_type=jax.ShapeDtypeStruct((NC, N_NODES, D_FEAT), jnp.float32),
        scratch_types=[
            pltpu.VMEM((NB, B_EDGE), jnp.int32),
            pltpu.VMEM((32, B_EDGE), jnp.int32),
            pltpu.VMEM((32, B_EDGE), jnp.int32),
            pltpu.VMEM((B_EDGE, D_FEAT), jnp.float32),
            pltpu.VMEM((B_EDGE, D_FEAT), jnp.float32),
            pltpu.VMEM_SHARED((N_NODES, D_FEAT), jnp.float32),
            pltpu.SemaphoreType.DMA,
            pltpu.SemaphoreType.DMA,
            pltpu.SemaphoreType.DMA,
        ],
    )
    def k(x_hbm, src_hbm, dst_hbm, zero_hbm, out_hbm,
          src_v, dst_a, dst_b, rows0, rows1, acc, gsem0, gsem1, ssem):
        cid = lax.axis_index("c")
        sid = lax.axis_index("s")
        wid = sid * NC + cid
        row0 = sid * ROWS_PER_TILE

        h_src = pltpu.async_copy(src_hbm.at[wid], src_v, gsem1)
        h_dst0 = pltpu.async_copy(dst_hbm.at[wid, pl.ds(0, 32)], dst_a, ssem)

        @pl.when(cid == 0)
        def _():
            pltpu.sync_copy(x_hbm.at[pl.ds(row0, ROWS_PER_TILE)],
                            acc.at[pl.ds(row0, ROWS_PER_TILE)])

            @pl.when(sid == NS - 1)
            def _():
                pltpu.sync_copy(
                    x_hbm.at[pl.ds(NS * ROWS_PER_TILE, ROWS_TAIL)],
                    acc.at[pl.ds(NS * ROWS_PER_TILE, ROWS_TAIL)])

        @pl.when(cid == 1)
        def _():
            pltpu.sync_copy(zero_hbm.at[pl.ds(0, ROWS_PER_TILE)],
                            acc.at[pl.ds(row0, ROWS_PER_TILE)])

            @pl.when(sid == NS - 1)
            def _():
                pltpu.sync_copy(
                    zero_hbm.at[pl.ds(0, ROWS_TAIL)],
                    acc.at[pl.ds(NS * ROWS_PER_TILE, ROWS_TAIL)])

        h_src.wait()
        h_dst0.wait()
        plsc.subcore_barrier()

        rows = (rows0, rows1)

        def run_block(base_g, dbuf, nu):
            hg = pltpu.async_copy(x_hbm.at[src_v.at[base_g]], rows0, gsem0)
            hs_prev = None
            for k in range(nu):
                hg.wait()
                if hs_prev is not None:
                    hs_prev.wait()
                hs = pltpu.async_copy(rows[k % 2],
                                      acc.at[dbuf.at[k]],
                                      ssem, add=True)
                if k + 1 < nu:
                    hg = pltpu.async_copy(x_hbm.at[src_v.at[base_g + k + 1]],
                                          rows[(k + 1) % 2], gsem0)
                hs_prev = hs
            hs_prev.wait()

        blocks = [(0, 32, dst_a), (32, 32, dst_b),
                  (64, 32, dst_a), (96, 29, dst_b)]
        for q, (base, nu, dbuf) in enumerate(blocks):
            h_next = None
            if q + 1 < len(blocks):
                nbase, nnu, nbuf = blocks[q + 1]
                h_next = pltpu.async_copy(
                    dst_hbm.at[wid, pl.ds(nbase, nnu)],
                    nbuf.at[pl.ds(0, nnu)], gsem1)
            run_block(base, dbuf, nu)
            if h_next is not None:
                h_next.wait()
        plsc.subcore_barrier()

        pltpu.sync_copy(
            acc.at[pl.ds(row0, ROWS_PER_TILE)],
            out_hbm.at[cid, pl.ds(row0, ROWS_PER_TILE)],
        )

        @pl.when(sid == NS - 1)
        def _():
            pltpu.sync_copy(
                acc.at[pl.ds(NS * ROWS_PER_TILE, ROWS_TAIL)],
                out_hbm.at[cid, pl.ds(NS * ROWS_PER_TILE, ROWS_TAIL)])

    return k(x, src3, dst3, zeros)


def _combine_tc(partials):
    def body(p_ref, o_ref):
        o_ref[...] = p_ref[0] + p_ref[1]

    rows = 1000
    grid = N_NODES // rows
    return pl.pallas_call(
        body,
        grid=(grid,),
        in_specs=[pl.BlockSpec((NC, rows, D_FEAT), lambda i: (0, i, 0))],
        out_specs=pl.BlockSpec((rows, D_FEAT), lambda i: (i, 0)),
        out_shape=jax.ShapeDtypeStruct((N_NODES, D_FEAT), jnp.float32),
    )(partials)


def kernel(x, edge_index):
    ei = edge_index.astype(jnp.int32)
    src3 = ei[0].reshape(NW, NB, B_EDGE)
    dst3 = ei[1].reshape(NW, NB, B_EDGE)
    zeros = jnp.zeros((ROWS_PER_TILE, D_FEAT), jnp.float32)
    partials = _segment_sum_sc(x, src3, dst3, zeros)
    return _combine_tc(partials)
